# Initial kernel scaffold; baseline (speedup 1.0000x reference)
import jax
import jax.numpy as jnp
from jax.experimental import pallas as pl


def kernel(x_ingredient, x_taste, edge_index, W_ing, b_ing, W_taste, b_taste, att_src, att_dst, W_k, b_k, q):
    raise NotImplementedError("write your pallas kernel here")


# speedup vs baseline: 13.4904x; 13.4904x over previous
import functools

import jax
import jax.numpy as jnp
import numpy as np
from jax import lax
from jax.experimental import pallas as pl
from jax.experimental.pallas import tpu as pltpu
from jax.experimental.pallas import tpu_sc as plsc

D = 128
HW = 72
H0 = 72
H1 = D - H0
NC, NS = 2, 16
NW = NC * NS
CH = 80
RB = 10


def _proj_body(xi_ref, xt_ref, wi_ref, bi_ref, wt_ref, bt_ref, avs_ref,
               avd_ref, h0_ref, h1_ref, als_ref, ald_ref):
    cdims = (((1,), (1,)), ((), ()))
    xs = lax.dot_general(xi_ref[...], wi_ref[...], cdims,
                         preferred_element_type=jnp.float32) + bi_ref[...]
    xd = lax.dot_general(xt_ref[...], wt_ref[...], cdims,
                         preferred_element_type=jnp.float32) + bt_ref[...]
    als_ref[...] = lax.dot_general(xs, avs_ref[...], cdims,
                                   preferred_element_type=jnp.float32)
    ald_ref[...] = lax.dot_general(xd, avd_ref[...], cdims,
                                   preferred_element_type=jnp.float32)
    h0_ref[...] = xs[:, :H0]
    n = xs.shape[0]
    lane = lax.broadcasted_iota(jnp.int32, (n, HW - H1), 1)
    extra = jnp.where(lane == 0, 1.0, 0.0).astype(jnp.float32)
    h1_ref[...] = jnp.concatenate([xs[:, H0:], extra], axis=1)


def _proj(x_ing, x_taste, w_ing, b_ing, w_taste, b_taste, av_s, av_d):
    n = x_ing.shape[0]
    blk = n // RB
    full = pl.BlockSpec((D, D), lambda i: (0, 0))
    vec = pl.BlockSpec((1, D), lambda i: (0, 0))
    rows = pl.BlockSpec((blk, D), lambda i: (i, 0))
    half = pl.BlockSpec((blk, HW), lambda i: (i, 0))
    return pl.pallas_call(
        _proj_body,
        grid=(RB,),
        in_specs=[rows, rows, full, vec, full, vec, vec, vec],
        out_specs=[
            half,
            half,
            pl.BlockSpec((blk, 1), lambda i: (i, 0)),
            pl.BlockSpec((blk, 1), lambda i: (i, 0)),
        ],
        out_shape=[
            jax.ShapeDtypeStruct((n, HW), jnp.float32),
            jax.ShapeDtypeStruct((n, HW), jnp.float32),
            jax.ShapeDtypeStruct((n, 1), jnp.float32),
            jax.ShapeDtypeStruct((n, 1), jnp.float32),
        ],
    )(x_ing, x_taste, w_ing, b_ing, w_taste, b_taste, av_s, av_d)


def _edge_body(h0_hbm, h1_hbm, src_hbm, dst_hbm, als_hbm, ald_hbm, out_hbm,
               as_v, ad_v, src_v, dst_v, w_v, rows_v, zbuf, acc, sem,
               *, n_pad, epw):
    c = lax.axis_index("c")
    s = lax.axis_index("s")
    wid = c * NS + s
    per_tile = n_pad // NS

    zero = jnp.zeros((16,), jnp.float32)

    def zfill(e, carry):
        for j in range(HW // 16):
            zbuf[e, pl.ds(j * 16, 16)] = zero
        zbuf[e, pl.ds(HW - 16, 16)] = zero
        return carry

    lax.fori_loop(0, zbuf.shape[0], zfill, 0)

    pltpu.sync_copy(als_hbm, as_v)
    pltpu.sync_copy(ald_hbm, ad_v)

    lane = lax.iota(jnp.int32, 16)

    for p, tab_hbm in ((0, h0_hbm), (1, h1_hbm)):
        pltpu.sync_copy(zbuf, acc.at[pl.ds(s * per_tile, per_tile)])
        plsc.subcore_barrier()

        def chunk(i, carry, tab_hbm=tab_hbm):
            eb = wid * epw + i * CH
            pltpu.sync_copy(src_hbm.at[pl.ds(eb, CH)], src_v)
            pltpu.sync_copy(dst_hbm.at[pl.ds(eb, CH)], dst_v)
            gather = pltpu.async_copy(tab_hbm.at[src_v], rows_v, sem)
            for g in range(CH // 16):
                sg = src_v[pl.ds(g * 16, 16)]
                dg = dst_v[pl.ds(g * 16, 16)]
                a = plsc.load_gather(as_v, [sg]) + plsc.load_gather(ad_v, [dg])
                a = jnp.where(a >= 0, a, 0.2 * a)
                w_v[pl.ds(g * 16, 16)] = jnp.exp(a)
            gather.wait()

            def scale(e, cc):
                wb = plsc.load_gather(w_v, [jnp.zeros((16,), jnp.int32) + e])
                for j in range(HW // 16):
                    rows_v[e, pl.ds(j * 16, 16)] = (
                        rows_v[e, pl.ds(j * 16, 16)] * wb)
                tail = jnp.where(lane < (16 - HW % 16), 1.0, wb)
                rows_v[e, pl.ds(HW - 16, 16)] = (
                    rows_v[e, pl.ds(HW - 16, 16)] * tail)
                return cc

            lax.fori_loop(0, CH, scale, 0)
            pltpu.sync_copy(rows_v, acc.at[dst_v], add=True)
            return carry

        lax.fori_loop(0, epw // CH, chunk, 0)
        plsc.subcore_barrier()
        pltpu.sync_copy(acc.at[pl.ds(s * per_tile, per_tile)],
                        out_hbm.at[c, p, pl.ds(s * per_tile, per_tile)])
        plsc.subcore_barrier()


def _edge(x_h0, x_h1, src, dst, als, ald):
    n_ing = x_h0.shape[0]
    n_taste = ald.shape[0]
    n_pad = ((n_taste + 8 * NS - 1) // (8 * NS)) * (8 * NS)
    e = src.shape[0]
    epw = e // NW
    mesh = plsc.VectorSubcoreMesh(core_axis_name="c", subcore_axis_name="s",
                                  num_cores=NC, num_subcores=NS)
    kfn = pl.kernel(
        functools.partial(_edge_body, n_pad=n_pad, epw=epw),
        out_type=jax.ShapeDtypeStruct((NC, 2, n_pad, HW), jnp.float32),
        mesh=mesh,
        scratch_types=[
            pltpu.VMEM((n_ing,), jnp.float32),
            pltpu.VMEM((n_taste,), jnp.float32),
            pltpu.VMEM((CH,), jnp.int32),
            pltpu.VMEM((CH,), jnp.int32),
            pltpu.VMEM((CH,), jnp.float32),
            pltpu.VMEM((CH, HW), jnp.float32),
            pltpu.VMEM((n_pad // NS, HW), jnp.float32),
            pltpu.VMEM_SHARED((n_pad, HW), jnp.float32),
            pltpu.SemaphoreType.DMA,
        ],
        compiler_params=pltpu.CompilerParams(use_tc_tiling_on_sc=False,
                                             needs_layout_passes=False),
    )
    return kfn(x_h0, x_h1, src, dst, als, ald)


def _finish_body(p_ref, o_ref):
    s0 = p_ref[0, 0] + p_ref[1, 0]
    s1 = p_ref[0, 1] + p_ref[1, 1]
    num = jnp.concatenate([s0, s1[:, :H1]], axis=1)
    den = s1[:, H1:H1 + 1]
    r = jnp.maximum(num / (den + 1e-16), 0.0)
    o_ref[...] = 0.5 * r * (1.0 + lax.erf(r * (1.0 / np.sqrt(2.0))))


def _finish(parts):
    n = parts.shape[2]
    blk = n // NS
    return pl.pallas_call(
        _finish_body,
        grid=(NS,),
        in_specs=[pl.BlockSpec((NC, 2, blk, HW), lambda i: (0, 0, i, 0))],
        out_specs=pl.BlockSpec((blk, D), lambda i: (i, 0)),
        out_shape=jax.ShapeDtypeStruct((n, D), jnp.float32),
    )(parts)


def kernel(x_ingredient, x_taste, edge_index, W_ing, b_ing, W_taste, b_taste,
           att_src, att_dst, W_k, b_k, q):
    del W_k, b_k, q
    src = edge_index[0]
    dst = edge_index[1]
    x_h0, x_h1, als, ald = _proj(
        x_ingredient, x_taste, W_ing, b_ing.reshape(1, D),
        W_taste, b_taste.reshape(1, D),
        att_src.reshape(1, D), att_dst.reshape(1, D))
    parts = _edge(x_h0, x_h1, src, dst, als.reshape(-1), ald.reshape(-1))
    return _finish(parts)[:x_taste.shape[0]]


# speedup vs baseline: 20.7392x; 1.5373x over previous
import functools

import jax
import jax.numpy as jnp
import numpy as np
from jax import lax
from jax.experimental import pallas as pl
from jax.experimental.pallas import tpu as pltpu
from jax.experimental.pallas import tpu_sc as plsc

D = 128
HW = 72
H0 = 72
H1 = D - H0
NC, NS = 2, 16
NW = NC * NS
CH = 80
RB = 10


def _proj_body(xi_ref, xt_ref, wi_ref, bi_ref, wt_ref, bt_ref, avs_ref,
               avd_ref, h0_ref, h1_ref, als_ref, ald_ref):
    cdims = (((1,), (1,)), ((), ()))
    xs = lax.dot_general(xi_ref[...], wi_ref[...], cdims,
                         preferred_element_type=jnp.float32) + bi_ref[...]
    xd = lax.dot_general(xt_ref[...], wt_ref[...], cdims,
                         preferred_element_type=jnp.float32) + bt_ref[...]
    als_ref[...] = lax.dot_general(xs, avs_ref[...], cdims,
                                   preferred_element_type=jnp.float32)
    ald_ref[...] = lax.dot_general(xd, avd_ref[...], cdims,
                                   preferred_element_type=jnp.float32)
    h0_ref[...] = xs[:, :H0]
    n = xs.shape[0]
    lane = lax.broadcasted_iota(jnp.int32, (n, HW - H1), 1)
    extra = jnp.where(lane == 0, 1.0, 0.0).astype(jnp.float32)
    h1_ref[...] = jnp.concatenate([xs[:, H0:], extra], axis=1)


def _proj(x_ing, x_taste, w_ing, b_ing, w_taste, b_taste, av_s, av_d):
    n = x_ing.shape[0]
    blk = n // RB
    full = pl.BlockSpec((D, D), lambda i: (0, 0))
    vec = pl.BlockSpec((1, D), lambda i: (0, 0))
    rows = pl.BlockSpec((blk, D), lambda i: (i, 0))
    half = pl.BlockSpec((blk, HW), lambda i: (i, 0))
    return pl.pallas_call(
        _proj_body,
        grid=(RB,),
        in_specs=[rows, rows, full, vec, full, vec, vec, vec],
        out_specs=[
            half,
            half,
            pl.BlockSpec((blk, 1), lambda i: (i, 0)),
            pl.BlockSpec((blk, 1), lambda i: (i, 0)),
        ],
        out_shape=[
            jax.ShapeDtypeStruct((n, HW), jnp.float32),
            jax.ShapeDtypeStruct((n, HW), jnp.float32),
            jax.ShapeDtypeStruct((n, 1), jnp.float32),
            jax.ShapeDtypeStruct((n, 1), jnp.float32),
        ],
    )(x_ing, x_taste, w_ing, b_ing, w_taste, b_taste, av_s, av_d)


def _edge_body(h0_hbm, h1_hbm, src_hbm, dst_hbm, als_hbm, ald_hbm, out_hbm,
               as_v, ad_v, src1d, dst1d, w1d, rows0, rows1, ssm0, ssm1,
               dsmall, acc, gsem0, gsem1, *, n_pad, epw):
    c = lax.axis_index("c")
    s = lax.axis_index("s")
    wid = c * NS + s
    per_tile = n_pad // NS
    nchunk = epw // CH
    rows = (rows0, rows1)
    ssm = (ssm0, ssm1)
    gsem = (gsem0, gsem1)

    zero = jnp.zeros((16,), jnp.float32)

    def zfill(e, carry):
        for j in range(HW // 16):
            rows0[e, pl.ds(j * 16, 16)] = zero
        rows0[e, pl.ds(HW - 16, 16)] = zero
        return carry

    def zero_my_slice():
        lax.fori_loop(0, CH, zfill, 0)
        base = s * per_tile
        for k in range(per_tile // CH):
            pltpu.sync_copy(rows0, acc.at[pl.ds(base + k * CH, CH)])
        rem = per_tile % CH
        if rem:
            pltpu.sync_copy(rows0.at[pl.ds(0, rem)],
                            acc.at[pl.ds(base + per_tile - rem, rem)])

    pltpu.sync_copy(src_hbm.at[wid], src1d)
    pltpu.sync_copy(dst_hbm.at[wid], dst1d)
    pltpu.sync_copy(als_hbm, as_v)
    pltpu.sync_copy(ald_hbm, ad_v)

    def wgrp(g, carry):
        sg = src1d[pl.ds(g * 16, 16)]
        dg = dst1d[pl.ds(g * 16, 16)]
        a = plsc.load_gather(as_v, [sg]) + plsc.load_gather(ad_v, [dg])
        a = jnp.where(a >= 0, a, 0.2 * a)
        w1d[pl.ds(g * 16, 16)] = jnp.exp(a)
        return carry

    lax.fori_loop(0, epw // 16, wgrp, 0)

    lane = lax.iota(jnp.int32, 16)
    z16 = jnp.zeros((16,), jnp.int32)

    def fill_idx(buf, src_ref, i):
        for g in range(CH // 16):
            buf[pl.ds(g * 16, 16)] = src_ref[pl.ds(i * CH + g * 16, 16)]

    for p, tab_hbm in ((0, h0_hbm), (1, h1_hbm)):
        zero_my_slice()
        plsc.subcore_barrier()

        def scale_scatter(i, b, tab_hbm=tab_hbm):
            rv = rows[b]

            def scale(e, cc):
                wb = plsc.load_gather(w1d, [z16 + (i * CH + e)])
                for j in range(HW // 16):
                    rv[e, pl.ds(j * 16, 16)] = rv[e, pl.ds(j * 16, 16)] * wb
                tail = jnp.where(lane < (16 - HW % 16), 1.0, wb)
                rv[e, pl.ds(HW - 16, 16)] = rv[e, pl.ds(HW - 16, 16)] * tail
                return cc

            lax.fori_loop(0, CH, scale, 0)
            fill_idx(dsmall, dst1d, i)
            pltpu.sync_copy(rv, acc.at[dsmall], add=True)

        def pair(k, carry, tab_hbm=tab_hbm):
            i0 = 2 * k
            i1 = 2 * k + 1
            fill_idx(ssm[0], src1d, i0)
            g0 = pltpu.async_copy(tab_hbm.at[ssm[0]], rows[0], gsem[0])
            fill_idx(ssm[1], src1d, i1)
            g1 = pltpu.async_copy(tab_hbm.at[ssm[1]], rows[1], gsem[1])
            g0.wait()
            scale_scatter(i0, 0)
            g1.wait()
            scale_scatter(i1, 1)
            return carry

        lax.fori_loop(0, nchunk // 2, pair, 0)
        if nchunk % 2:
            i_last = nchunk - 1
            fill_idx(ssm[0], src1d, i_last)
            pltpu.async_copy(tab_hbm.at[ssm[0]], rows[0], gsem[0]).wait()
            scale_scatter(i_last, 0)

        plsc.subcore_barrier()
        pltpu.sync_copy(acc.at[pl.ds(s * per_tile, per_tile)],
                        out_hbm.at[c, p, pl.ds(s * per_tile, per_tile)])
        plsc.subcore_barrier()


def _edge(x_h0, x_h1, src, dst, als, ald):
    n_ing = x_h0.shape[0]
    n_taste = ald.shape[0]
    n_pad = ((n_taste + 8 * NS - 1) // (8 * NS)) * (8 * NS)
    e = src.shape[0]
    epw = e // NW
    nchunk = epw // CH
    mesh = plsc.VectorSubcoreMesh(core_axis_name="c", subcore_axis_name="s",
                                  num_cores=NC, num_subcores=NS)
    kfn = pl.kernel(
        functools.partial(_edge_body, n_pad=n_pad, epw=epw),
        out_type=jax.ShapeDtypeStruct((NC, 2, n_pad, HW), jnp.float32),
        mesh=mesh,
        scratch_types=[
            pltpu.VMEM((n_ing,), jnp.float32),
            pltpu.VMEM((n_taste,), jnp.float32),
            pltpu.VMEM((epw,), jnp.int32),
            pltpu.VMEM((epw,), jnp.int32),
            pltpu.VMEM((epw,), jnp.float32),
            pltpu.VMEM((CH, HW), jnp.float32),
            pltpu.VMEM((CH, HW), jnp.float32),
            pltpu.VMEM((CH,), jnp.int32),
            pltpu.VMEM((CH,), jnp.int32),
            pltpu.VMEM((CH,), jnp.int32),
            pltpu.VMEM_SHARED((n_pad, HW), jnp.float32),
            pltpu.SemaphoreType.DMA,
            pltpu.SemaphoreType.DMA,
        ],
        compiler_params=pltpu.CompilerParams(use_tc_tiling_on_sc=False,
                                             needs_layout_passes=False),
    )
    return kfn(x_h0, x_h1, src.reshape(NW, epw), dst.reshape(NW, epw),
               als, ald)


def _finish_body(p_ref, o_ref):
    s0 = p_ref[0, 0] + p_ref[1, 0]
    s1 = p_ref[0, 1] + p_ref[1, 1]
    num = jnp.concatenate([s0, s1[:, :H1]], axis=1)
    den = s1[:, H1:H1 + 1]
    r = jnp.maximum(num / (den + 1e-16), 0.0)
    o_ref[...] = 0.5 * r * (1.0 + lax.erf(r * (1.0 / np.sqrt(2.0))))


def _finish(parts):
    n = parts.shape[2]
    blk = n // NS
    return pl.pallas_call(
        _finish_body,
        grid=(NS,),
        in_specs=[pl.BlockSpec((NC, 2, blk, HW), lambda i: (0, 0, i, 0))],
        out_specs=pl.BlockSpec((blk, D), lambda i: (i, 0)),
        out_shape=jax.ShapeDtypeStruct((n, D), jnp.float32),
    )(parts)


def kernel(x_ingredient, x_taste, edge_index, W_ing, b_ing, W_taste, b_taste,
           att_src, att_dst, W_k, b_k, q):
    del W_k, b_k, q
    src = edge_index[0]
    dst = edge_index[1]
    x_h0, x_h1, als, ald = _proj(
        x_ingredient, x_taste, W_ing, b_ing.reshape(1, D),
        W_taste, b_taste.reshape(1, D),
        att_src.reshape(1, D), att_dst.reshape(1, D))
    parts = _edge(x_h0, x_h1, src, dst, als.reshape(-1), ald.reshape(-1))
    return _finish(parts)[:x_taste.shape[0]]


# speedup vs baseline: 23.9593x; 1.1553x over previous
import functools

import jax
import jax.numpy as jnp
import numpy as np
from jax import lax
from jax.experimental import pallas as pl
from jax.experimental.pallas import tpu as pltpu
from jax.experimental.pallas import tpu_sc as plsc

D = 128
HW = 72
H0 = 72
H1 = D - H0
NC, NS = 2, 16
NW = NC * NS
CH = 80
RB = 10


def _proj_body(xi_ref, xt_ref, wi_ref, bi_ref, wt_ref, bt_ref, avs_ref,
               avd_ref, h0_ref, h1_ref, als_ref, ald_ref):
    cdims = (((1,), (1,)), ((), ()))
    xs = lax.dot_general(xi_ref[...], wi_ref[...], cdims,
                         preferred_element_type=jnp.float32) + bi_ref[...]
    xd = lax.dot_general(xt_ref[...], wt_ref[...], cdims,
                         preferred_element_type=jnp.float32) + bt_ref[...]
    als_ref[...] = lax.dot_general(xs, avs_ref[...], cdims,
                                   preferred_element_type=jnp.float32)
    ald_ref[...] = lax.dot_general(xd, avd_ref[...], cdims,
                                   preferred_element_type=jnp.float32)
    h0_ref[...] = xs[:, :H0]
    n = xs.shape[0]
    lane = lax.broadcasted_iota(jnp.int32, (n, HW - H1), 1)
    extra = jnp.where(lane == 0, 1.0, 0.0).astype(jnp.float32)
    h1_ref[...] = jnp.concatenate([xs[:, H0:], extra], axis=1)


def _proj(x_ing, x_taste, w_ing, b_ing, w_taste, b_taste, av_s, av_d):
    n = x_ing.shape[0]
    blk = n // RB
    full = pl.BlockSpec((D, D), lambda i: (0, 0))
    vec = pl.BlockSpec((1, D), lambda i: (0, 0))
    rows = pl.BlockSpec((blk, D), lambda i: (i, 0))
    half = pl.BlockSpec((blk, HW), lambda i: (i, 0))
    return pl.pallas_call(
        _proj_body,
        grid=(RB,),
        in_specs=[rows, rows, full, vec, full, vec, vec, vec],
        out_specs=[
            half,
            half,
            pl.BlockSpec((blk, 1), lambda i: (i, 0)),
            pl.BlockSpec((blk, 1), lambda i: (i, 0)),
        ],
        out_shape=[
            jax.ShapeDtypeStruct((n, HW), jnp.float32),
            jax.ShapeDtypeStruct((n, HW), jnp.float32),
            jax.ShapeDtypeStruct((n, 1), jnp.float32),
            jax.ShapeDtypeStruct((n, 1), jnp.float32),
        ],
    )(x_ing, x_taste, w_ing, b_ing, w_taste, b_taste, av_s, av_d)


def _edge_body(h0_hbm, h1_hbm, src_hbm, dst_hbm, als_hbm, ald_hbm, out_hbm,
               as_v, ad_v, src1d, dst1d, w1d, rows0, rows1, ssm0, ssm1,
               dsm0, dsm1, acc, gsem0, gsem1, ssem0, ssem1, *, n_pad, epw):
    c = lax.axis_index("c")
    s = lax.axis_index("s")
    wid = c * NS + s
    per_tile = n_pad // NS
    nchunk = epw // CH
    rows = (rows0, rows1)
    ssm = (ssm0, ssm1)
    dsm = (dsm0, dsm1)
    gsem = (gsem0, gsem1)
    ssem = (ssem0, ssem1)

    zero = jnp.zeros((16,), jnp.float32)

    def zfill(e, carry):
        for j in range(HW // 16):
            rows0[e, pl.ds(j * 16, 16)] = zero
        rows0[e, pl.ds(HW - 16, 16)] = zero
        return carry

    def zero_my_slice():
        lax.fori_loop(0, CH, zfill, 0)
        base = s * per_tile
        for k in range(per_tile // CH):
            pltpu.sync_copy(rows0, acc.at[pl.ds(base + k * CH, CH)])
        rem = per_tile % CH
        if rem:
            pltpu.sync_copy(rows0.at[pl.ds(0, rem)],
                            acc.at[pl.ds(base + per_tile - rem, rem)])

    pltpu.sync_copy(src_hbm.at[wid], src1d)
    pltpu.sync_copy(dst_hbm.at[wid], dst1d)
    pltpu.sync_copy(als_hbm, as_v)
    pltpu.sync_copy(ald_hbm, ad_v)

    def wgrp(g, carry):
        sg = src1d[pl.ds(g * 16, 16)]
        dg = dst1d[pl.ds(g * 16, 16)]
        a = plsc.load_gather(as_v, [sg]) + plsc.load_gather(ad_v, [dg])
        a = jnp.where(a >= 0, a, 0.2 * a)
        w1d[pl.ds(g * 16, 16)] = jnp.exp(a)
        return carry

    lax.fori_loop(0, epw // 16, wgrp, 0)

    lane = lax.iota(jnp.int32, 16)
    z16 = jnp.zeros((16,), jnp.int32)

    def fill_idx(buf, src_ref, i):
        for g in range(CH // 16):
            buf[pl.ds(g * 16, 16)] = src_ref[pl.ds(i * CH + g * 16, 16)]

    for p, tab_hbm in ((0, h0_hbm), (1, h1_hbm)):
        zero_my_slice()
        plsc.subcore_barrier()

        def scale_scatter(i, b, tab_hbm=tab_hbm):
            rv = rows[b]

            def scale(e, cc):
                wb = plsc.load_gather(w1d, [z16 + (i * CH + e)])
                for j in range(HW // 16):
                    rv[e, pl.ds(j * 16, 16)] = rv[e, pl.ds(j * 16, 16)] * wb
                tail = jnp.where(lane < (16 - HW % 16), 1.0, wb)
                rv[e, pl.ds(HW - 16, 16)] = rv[e, pl.ds(HW - 16, 16)] * tail
                return cc

            lax.fori_loop(0, CH, scale, 0)
            fill_idx(dsm[b], dst1d, i)
            pltpu.async_copy(rv, acc.at[dsm[b]], ssem[b], add=True)

        def drain_scatter(b):
            pltpu.make_async_copy(rows[b], acc.at[dsm[b]], ssem[b]).wait()

        def pair(k, carry, tab_hbm=tab_hbm, first=False):
            i0 = 2 * k
            i1 = 2 * k + 1
            if not first:
                drain_scatter(0)
            fill_idx(ssm[0], src1d, i0)
            g0 = pltpu.async_copy(tab_hbm.at[ssm[0]], rows[0], gsem[0])
            if not first:
                drain_scatter(1)
            fill_idx(ssm[1], src1d, i1)
            g1 = pltpu.async_copy(tab_hbm.at[ssm[1]], rows[1], gsem[1])
            g0.wait()
            scale_scatter(i0, 0)
            g1.wait()
            scale_scatter(i1, 1)
            return carry

        pair(0, 0, first=True)
        lax.fori_loop(1, nchunk // 2, pair, 0)
        drain_scatter(0)
        if nchunk % 2:
            i_last = nchunk - 1
            fill_idx(ssm[0], src1d, i_last)
            pltpu.async_copy(tab_hbm.at[ssm[0]], rows[0], gsem[0]).wait()
            scale_scatter(i_last, 0)
            drain_scatter(0)
        drain_scatter(1)

        plsc.subcore_barrier()
        pltpu.sync_copy(acc.at[pl.ds(s * per_tile, per_tile)],
                        out_hbm.at[c, p, pl.ds(s * per_tile, per_tile)])
        plsc.subcore_barrier()


def _edge(x_h0, x_h1, src, dst, als, ald):
    n_ing = x_h0.shape[0]
    n_taste = ald.shape[0]
    n_pad = ((n_taste + 8 * NS - 1) // (8 * NS)) * (8 * NS)
    e = src.shape[0]
    epw = e // NW
    nchunk = epw // CH
    mesh = plsc.VectorSubcoreMesh(core_axis_name="c", subcore_axis_name="s",
                                  num_cores=NC, num_subcores=NS)
    kfn = pl.kernel(
        functools.partial(_edge_body, n_pad=n_pad, epw=epw),
        out_type=jax.ShapeDtypeStruct((NC, 2, n_pad, HW), jnp.float32),
        mesh=mesh,
        scratch_types=[
            pltpu.VMEM((n_ing,), jnp.float32),
            pltpu.VMEM((n_taste,), jnp.float32),
            pltpu.VMEM((epw,), jnp.int32),
            pltpu.VMEM((epw,), jnp.int32),
            pltpu.VMEM((epw,), jnp.float32),
            pltpu.VMEM((CH, HW), jnp.float32),
            pltpu.VMEM((CH, HW), jnp.float32),
            pltpu.VMEM((CH,), jnp.int32),
            pltpu.VMEM((CH,), jnp.int32),
            pltpu.VMEM((CH,), jnp.int32),
            pltpu.VMEM((CH,), jnp.int32),
            pltpu.VMEM_SHARED((n_pad, HW), jnp.float32),
            pltpu.SemaphoreType.DMA,
            pltpu.SemaphoreType.DMA,
            pltpu.SemaphoreType.DMA,
            pltpu.SemaphoreType.DMA,
        ],
        compiler_params=pltpu.CompilerParams(use_tc_tiling_on_sc=False,
                                             needs_layout_passes=False),
    )
    return kfn(x_h0, x_h1, src.reshape(NW, epw), dst.reshape(NW, epw),
               als, ald)


def _finish_body(p_ref, o_ref):
    s0 = p_ref[0, 0] + p_ref[1, 0]
    s1 = p_ref[0, 1] + p_ref[1, 1]
    num = jnp.concatenate([s0, s1[:, :H1]], axis=1)
    den = s1[:, H1:H1 + 1]
    r = jnp.maximum(num / (den + 1e-16), 0.0)
    o_ref[...] = 0.5 * r * (1.0 + lax.erf(r * (1.0 / np.sqrt(2.0))))


def _finish(parts):
    n = parts.shape[2]
    blk = n // NS
    return pl.pallas_call(
        _finish_body,
        grid=(NS,),
        in_specs=[pl.BlockSpec((NC, 2, blk, HW), lambda i: (0, 0, i, 0))],
        out_specs=pl.BlockSpec((blk, D), lambda i: (i, 0)),
        out_shape=jax.ShapeDtypeStruct((n, D), jnp.float32),
    )(parts)


def kernel(x_ingredient, x_taste, edge_index, W_ing, b_ing, W_taste, b_taste,
           att_src, att_dst, W_k, b_k, q):
    del W_k, b_k, q
    src = edge_index[0]
    dst = edge_index[1]
    x_h0, x_h1, als, ald = _proj(
        x_ingredient, x_taste, W_ing, b_ing.reshape(1, D),
        W_taste, b_taste.reshape(1, D),
        att_src.reshape(1, D), att_dst.reshape(1, D))
    parts = _edge(x_h0, x_h1, src, dst, als.reshape(-1), ald.reshape(-1))
    return _finish(parts)[:x_taste.shape[0]]
